# baseline (speedup 1.0000x reference)
import functools
import math

import jax
import jax.numpy as jnp
from jax import lax
from jax.experimental import pallas as pl
from jax.experimental.pallas import tpu as pltpu
from jax.experimental.pallas import tpu_sc as plsc

N = 10000
E = 160000
D = 256
G = 64
NC = 2
NS = 16
CHUNK = 40
EPW = E // NS
NCHUNK = EPW // CHUNK
NPAD = 10240
NHALF = 5120
ACC_ROWS = 5248
DUMP = 5120
ZSTRIPE = ACC_ROWS // NS
OSTRIPE = NHALF // NS
ZROWS = 8

_BN_INV = float(1.0 / math.sqrt(1.0 + 1e-5))


def _sc_edge_body(h2, ee2, ridx, cidx2, agg2, idxr_v, idxc_v, hbuf, eebuf,
                  zbuf, acc, gsem, ssem):
    c = lax.axis_index("c")
    s = lax.axis_index("s")

    pltpu.sync_copy(ridx.at[s], idxr_v)

    z16 = jnp.zeros((16,), jnp.float32)
    for i in range(ZROWS):
        for d in range(8):
            zbuf[i, pl.ds(16 * d, 16)] = z16

    for p in range(2):
        pltpu.sync_copy(cidx2.at[p, s], idxc_v)
        for j in range(ZSTRIPE // ZROWS):
            z0 = pl.multiple_of(s * ZSTRIPE + j * ZROWS, ZROWS)
            pltpu.sync_copy(zbuf, acc.at[pl.ds(z0, ZROWS)])
        plsc.subcore_barrier()

        def chunk_body(j, carry):
            pltpu.async_copy(h2.at[c].at[idxr_v.at[j]], hbuf, gsem).wait()
            e0 = pl.multiple_of(s * EPW + j * CHUNK, 8)
            pltpu.sync_copy(ee2.at[c, pl.ds(e0, CHUNK)], eebuf)
            for r in range(CHUNK):
                for d in range(8):
                    sl = pl.ds(16 * d, 16)
                    hbuf[r, sl] = jnp.maximum(hbuf[r, sl] + eebuf[r, sl], 0.0)
            pltpu.async_copy(hbuf, acc.at[idxc_v.at[j]], ssem, add=True).wait()
            return carry

        lax.fori_loop(0, NCHUNK, chunk_body, 0)
        plsc.subcore_barrier()

        r0 = pl.multiple_of(s * OSTRIPE, 8)
        pltpu.sync_copy(acc.at[pl.ds(r0, OSTRIPE)],
                        agg2.at[c, pl.ds(p * NHALF + r0, OSTRIPE)])
        plsc.subcore_barrier()


def _sc_edge(hin2, ee2, ridx, cidx2):
    mesh = plsc.VectorSubcoreMesh(core_axis_name="c", subcore_axis_name="s",
                                  num_cores=NC, num_subcores=NS)
    f = pl.kernel(
        _sc_edge_body,
        out_type=jax.ShapeDtypeStruct((NC, NPAD, 128), jnp.float32),
        mesh=mesh,
        scratch_types=[
            pltpu.VMEM((NCHUNK, CHUNK), jnp.int32),
            pltpu.VMEM((NCHUNK, CHUNK), jnp.int32),
            pltpu.VMEM((CHUNK, 128), jnp.float32),
            pltpu.VMEM((CHUNK, 128), jnp.float32),
            pltpu.VMEM((ZROWS, 128), jnp.float32),
            pltpu.VMEM_SHARED((ACC_ROWS, 128), jnp.float32),
            pltpu.SemaphoreType.DMA,
            pltpu.SemaphoreType.DMA,
        ],
    )
    return f(hin2, ee2, ridx, cidx2)


_BN_ROWS = 2000


def _pre_body(h2, vn, batch, hin2, pooled):
    i = pl.program_id(0)
    h = jnp.concatenate([h2[0], h2[1]], axis=1)
    oh = (batch[...] == lax.broadcasted_iota(jnp.int32, (1, G), 1)
          ).astype(jnp.float32)
    hin = h + jnp.dot(oh, vn[...], preferred_element_type=jnp.float32)
    hin2[0] = hin[:, :128]
    hin2[1] = hin[:, 128:]
    p = lax.dot_general(oh, hin, (((0,), (0,)), ((), ())),
                        preferred_element_type=jnp.float32)

    @pl.when(i == 0)
    def _():
        pooled[...] = p

    @pl.when(i > 0)
    def _():
        pooled[...] += p


def _tc_pre(h2, vn, batch2):
    return pl.pallas_call(
        _pre_body,
        grid=(N // _BN_ROWS,),
        in_specs=[
            pl.BlockSpec((NC, _BN_ROWS, 128), lambda i: (0, i, 0)),
            pl.BlockSpec((G, D), lambda i: (0, 0)),
            pl.BlockSpec((_BN_ROWS, 1), lambda i: (i, 0)),
        ],
        out_specs=[
            pl.BlockSpec((NC, _BN_ROWS, 128), lambda i: (0, i, 0)),
            pl.BlockSpec((G, D), lambda i: (0, 0)),
        ],
        out_shape=[
            jax.ShapeDtypeStruct((NC, N, 128), jnp.float32),
            jax.ShapeDtypeStruct((G, D), jnp.float32),
        ],
    )(h2, vn, batch2)


_EE_ROWS = 4000


def _ee_body(ea, w, b, out):
    t = jnp.dot(ea[...], w[...], preferred_element_type=jnp.float32) + b[...]
    out[0] = t[:, :128]
    out[1] = t[:, 128:]


def _tc_ee(ea8, w8, b):
    return pl.pallas_call(
        _ee_body,
        grid=(E // _EE_ROWS,),
        in_specs=[
            pl.BlockSpec((_EE_ROWS, 8), lambda i: (i, 0)),
            pl.BlockSpec((8, D), lambda i: (0, 0)),
            pl.BlockSpec((1, D), lambda i: (0, 0)),
        ],
        out_specs=pl.BlockSpec((NC, _EE_ROWS, 128), lambda i: (0, i, 0)),
        out_shape=jax.ShapeDtypeStruct((NC, E, 128), jnp.float32),
    )(ea8, w8, b)


def _post_body(last, hin2, agg2, eps, w1, b1, g1, bb1, w2, b2, g2, bb2, out):
    hin = jnp.concatenate([hin2[0], hin2[1]], axis=1)
    agg = jnp.concatenate([agg2[0], agg2[1]], axis=1)
    t = (1.0 + eps[0, 0]) * hin + agg
    t = jnp.dot(t, w1[...], preferred_element_type=jnp.float32) + b1[...]
    t = t * (_BN_INV * g1[...]) + bb1[...]
    t = jnp.maximum(t, 0.0)
    t = jnp.dot(t, w2[...], preferred_element_type=jnp.float32) + b2[...]
    t = t * (_BN_INV * g2[...]) + bb2[...]
    if not last:
        t = jnp.maximum(t, 0.0)
        out[0] = t[:, :128]
        out[1] = t[:, 128:]
    else:
        out[...] = t


def _tc_post(hin2, agg2, eps_l, w1, b1, g1, bb1, w2, b2, g2, bb2, last):
    if last:
        out_spec = pl.BlockSpec((_BN_ROWS, D), lambda i: (i, 0))
        out_shape = jax.ShapeDtypeStruct((N, D), jnp.float32)
    else:
        out_spec = pl.BlockSpec((NC, _BN_ROWS, 128), lambda i: (0, i, 0))
        out_shape = jax.ShapeDtypeStruct((NC, N, 128), jnp.float32)
    return pl.pallas_call(
        functools.partial(_post_body, last),
        grid=(N // _BN_ROWS,),
        in_specs=[
            pl.BlockSpec((NC, _BN_ROWS, 128), lambda i: (0, i, 0)),
            pl.BlockSpec((NC, _BN_ROWS, 128), lambda i: (0, i, 0)),
            pl.BlockSpec((1, 1), lambda i: (0, 0)),
            pl.BlockSpec((D, 2 * D), lambda i: (0, 0)),
            pl.BlockSpec((1, 2 * D), lambda i: (0, 0)),
            pl.BlockSpec((1, 2 * D), lambda i: (0, 0)),
            pl.BlockSpec((1, 2 * D), lambda i: (0, 0)),
            pl.BlockSpec((2 * D, D), lambda i: (0, 0)),
            pl.BlockSpec((1, D), lambda i: (0, 0)),
            pl.BlockSpec((1, D), lambda i: (0, 0)),
            pl.BlockSpec((1, D), lambda i: (0, 0)),
        ],
        out_specs=out_spec,
        out_shape=out_shape,
    )(hin2, agg2, eps_l, w1, b1, g1, bb1, w2, b2, g2, bb2)


def _vn_body(pooled, vn, w1, b1, g1, bb1, w2, b2, g2, bb2, out):
    v = pooled[...] + vn[...]
    v = jnp.dot(v, w1[...], preferred_element_type=jnp.float32) + b1[...]
    v = v * (_BN_INV * g1[...]) + bb1[...]
    v = jnp.maximum(v, 0.0)
    v = jnp.dot(v, w2[...], preferred_element_type=jnp.float32) + b2[...]
    v = v * (_BN_INV * g2[...]) + bb2[...]
    out[...] = jnp.maximum(v, 0.0)


def _tc_vn(pooled, vn, w1, b1, g1, bb1, w2, b2, g2, bb2):
    return pl.pallas_call(
        _vn_body,
        out_shape=jax.ShapeDtypeStruct((G, D), jnp.float32),
    )(pooled, vn, w1, b1, g1, bb1, w2, b2, g2, bb2)


def kernel(x, edge_attr, edge_W, edge_b, eps, mlp_W1, mlp_b1, mlp_bn_g,
           mlp_bn_b, mlp_W2, mlp_b2, bn_g, bn_b, vn_emb, vW1, vb1, vbn1_g,
           vbn1_b, vW2, vb2, vbn2_g, vbn2_b, edge_index, batch):
    L = edge_W.shape[0]
    h2 = jnp.stack([x[:, :128], x[:, 128:]], axis=0)
    ea8 = jnp.pad(edge_attr, ((0, 0), (0, 1)))
    w8 = jnp.pad(edge_W, ((0, 0), (0, 1), (0, 0)))
    ridx = edge_index[0].reshape(NS, NCHUNK, CHUNK)
    col = edge_index[1]
    cidx2 = jnp.stack([jnp.where(col < NHALF, col, DUMP),
                       jnp.where(col >= NHALF, col - NHALF, DUMP)],
                      axis=0).reshape(2, NS, NCHUNK, CHUNK)
    batch2 = batch.reshape(N, 1)
    vn = jnp.tile(vn_emb[None, :], (G, 1))

    def r1(a):
        return a.reshape(1, -1)

    out = None
    for l in range(L):
        hin2, pooled = _tc_pre(h2, vn, batch2)
        ee2 = _tc_ee(ea8, w8[l], r1(edge_b[l]))
        agg2 = _sc_edge(hin2, ee2, ridx, cidx2)
        last = l == L - 1
        res = _tc_post(hin2, agg2, eps[l].reshape(1, 1),
                       mlp_W1[l], r1(mlp_b1[l]), r1(mlp_bn_g[l]),
                       r1(mlp_bn_b[l]), mlp_W2[l], r1(mlp_b2[l]),
                       r1(bn_g[l]), r1(bn_b[l]), last)
        if last:
            out = res
        else:
            h2 = res
            vn = _tc_vn(pooled, vn, vW1[l], r1(vb1[l]), r1(vbn1_g[l]),
                        r1(vbn1_b[l]), vW2[l], r1(vb2[l]), r1(vbn2_g[l]),
                        r1(vbn2_b[l]))
    return out


# speedup vs baseline: 2.5009x; 2.5009x over previous
import functools
import math

import jax
import jax.numpy as jnp
from jax import lax
from jax.experimental import pallas as pl
from jax.experimental.pallas import tpu as pltpu
from jax.experimental.pallas import tpu_sc as plsc

N = 10000
E = 160000
D = 256
G = 64
NC = 2
NS = 16
CHUNK = 80
EPW = E // NS
NCHUNK = EPW // CHUNK
NPAD = 10240
NHALF = 5120
ACC_ROWS = 5248
DUMP = 5120
ZSTRIPE = ACC_ROWS // NS
OSTRIPE = NHALF // NS
ZROWS = 8

_BN_INV = float(1.0 / math.sqrt(1.0 + 1e-5))


def _sc_edge_body(h2, ee2, ridx, cidx2, agg2, idxr_v, idxc_v,
                  hbuf0, hbuf1, eebuf0, eebuf1, zbuf, acc,
                  gsem0, gsem1, esem0, esem1, ssem0, ssem1):
    c = lax.axis_index("c")
    s = lax.axis_index("s")
    hbufs = (hbuf0, hbuf1)
    eebufs = (eebuf0, eebuf1)
    gsems = (gsem0, gsem1)
    esems = (esem0, esem1)
    ssems = (ssem0, ssem1)

    pltpu.sync_copy(ridx.at[s], idxr_v)

    z16 = jnp.zeros((16,), jnp.float32)
    for i in range(ZROWS):
        for d in range(8):
            zbuf[i, pl.ds(16 * d, 16)] = z16

    def issue(j, k):
        pltpu.async_copy(h2.at[c].at[idxr_v.at[j]], hbufs[k], gsems[k])
        e0 = pl.multiple_of(s * EPW + j * CHUNK, 8)
        pltpu.async_copy(ee2.at[c, pl.ds(e0, CHUNK)], eebufs[k], esems[k])

    def process(j, k):
        pltpu.make_async_copy(h2.at[c].at[idxr_v.at[j]], hbufs[k],
                              gsems[k]).wait()
        e0 = pl.multiple_of(s * EPW + j * CHUNK, 8)
        pltpu.make_async_copy(ee2.at[c, pl.ds(e0, CHUNK)], eebufs[k],
                              esems[k]).wait()
        hb, eb = hbufs[k], eebufs[k]

        def row_body(r4, carry):
            for u in range(4):
                r = r4 * 4 + u
                for d in range(8):
                    sl = pl.ds(16 * d, 16)
                    hb[r, sl] = jnp.maximum(hb[r, sl] + eb[r, sl], 0.0)
            return carry

        lax.fori_loop(0, CHUNK // 4, row_body, 0)
        pltpu.async_copy(hb, acc.at[idxc_v.at[j]], ssems[k], add=True)

    def drain_scatter(j, k):
        pltpu.make_async_copy(hbufs[k], acc.at[idxc_v.at[j]],
                              ssems[k]).wait()

    for p in range(2):
        pltpu.sync_copy(cidx2.at[p, s], idxc_v)
        for j in range(ZSTRIPE // ZROWS):
            z0 = pl.multiple_of(s * ZSTRIPE + j * ZROWS, ZROWS)
            pltpu.sync_copy(zbuf, acc.at[pl.ds(z0, ZROWS)])
        plsc.subcore_barrier()

        issue(0, 0)
        issue(1, 1)

        def pair_body(i, carry):
            j = i * 2
            process(j, 0)
            process(j + 1, 1)
            drain_scatter(j, 0)

            @pl.when(j + 2 < NCHUNK - 1)
            def _():
                issue(j + 2, 0)

            drain_scatter(j + 1, 1)

            @pl.when(j + 3 < NCHUNK - 1)
            def _():
                issue(j + 3, 1)

            return carry

        lax.fori_loop(0, (NCHUNK - 1) // 2, pair_body, 0)
        issue(NCHUNK - 1, 0)
        process(NCHUNK - 1, 0)
        drain_scatter(NCHUNK - 1, 0)
        plsc.subcore_barrier()

        r0 = pl.multiple_of(s * OSTRIPE, 8)
        pltpu.sync_copy(acc.at[pl.ds(r0, OSTRIPE)],
                        agg2.at[c, pl.ds(p * NHALF + r0, OSTRIPE)])
        plsc.subcore_barrier()


def _sc_edge(hin2, ee2, ridx, cidx2):
    mesh = plsc.VectorSubcoreMesh(core_axis_name="c", subcore_axis_name="s",
                                  num_cores=NC, num_subcores=NS)
    f = pl.kernel(
        _sc_edge_body,
        out_type=jax.ShapeDtypeStruct((NC, NPAD, 128), jnp.float32),
        mesh=mesh,
        scratch_types=[
            pltpu.VMEM((NCHUNK, CHUNK), jnp.int32),
            pltpu.VMEM((NCHUNK, CHUNK), jnp.int32),
            pltpu.VMEM((CHUNK, 128), jnp.float32),
            pltpu.VMEM((CHUNK, 128), jnp.float32),
            pltpu.VMEM((CHUNK, 128), jnp.float32),
            pltpu.VMEM((CHUNK, 128), jnp.float32),
            pltpu.VMEM((ZROWS, 128), jnp.float32),
            pltpu.VMEM_SHARED((ACC_ROWS, 128), jnp.float32),
        ] + [pltpu.SemaphoreType.DMA] * 6,
    )
    return f(hin2, ee2, ridx, cidx2)


_BN_ROWS = 2000


def _pre_body(h2, vn, batch, hin2, pooled):
    i = pl.program_id(0)
    h = jnp.concatenate([h2[0], h2[1]], axis=1)
    oh = (batch[...] == lax.broadcasted_iota(jnp.int32, (1, G), 1)
          ).astype(jnp.float32)
    hin = h + jnp.dot(oh, vn[...], preferred_element_type=jnp.float32)
    hin2[0] = hin[:, :128]
    hin2[1] = hin[:, 128:]
    p = lax.dot_general(oh, hin, (((0,), (0,)), ((), ())),
                        preferred_element_type=jnp.float32)

    @pl.when(i == 0)
    def _():
        pooled[...] = p

    @pl.when(i > 0)
    def _():
        pooled[...] += p


def _tc_pre(h2, vn, batch2):
    return pl.pallas_call(
        _pre_body,
        grid=(N // _BN_ROWS,),
        in_specs=[
            pl.BlockSpec((NC, _BN_ROWS, 128), lambda i: (0, i, 0)),
            pl.BlockSpec((G, D), lambda i: (0, 0)),
            pl.BlockSpec((_BN_ROWS, 1), lambda i: (i, 0)),
        ],
        out_specs=[
            pl.BlockSpec((NC, _BN_ROWS, 128), lambda i: (0, i, 0)),
            pl.BlockSpec((G, D), lambda i: (0, 0)),
        ],
        out_shape=[
            jax.ShapeDtypeStruct((NC, N, 128), jnp.float32),
            jax.ShapeDtypeStruct((G, D), jnp.float32),
        ],
    )(h2, vn, batch2)


_EE_ROWS = 4000


def _ee_body(ea, w, b, out):
    t = jnp.dot(ea[...], w[...], preferred_element_type=jnp.float32) + b[...]
    out[0] = t[:, :128]
    out[1] = t[:, 128:]


def _tc_ee(ea8, w8, b):
    return pl.pallas_call(
        _ee_body,
        grid=(E // _EE_ROWS,),
        in_specs=[
            pl.BlockSpec((_EE_ROWS, 8), lambda i: (i, 0)),
            pl.BlockSpec((8, D), lambda i: (0, 0)),
            pl.BlockSpec((1, D), lambda i: (0, 0)),
        ],
        out_specs=pl.BlockSpec((NC, _EE_ROWS, 128), lambda i: (0, i, 0)),
        out_shape=jax.ShapeDtypeStruct((NC, E, 128), jnp.float32),
    )(ea8, w8, b)


def _post_body(last, hin2, agg2, eps, w1, b1, g1, bb1, w2, b2, g2, bb2, out):
    hin = jnp.concatenate([hin2[0], hin2[1]], axis=1)
    agg = jnp.concatenate([agg2[0], agg2[1]], axis=1)
    t = (1.0 + eps[0, 0]) * hin + agg
    t = jnp.dot(t, w1[...], preferred_element_type=jnp.float32) + b1[...]
    t = t * (_BN_INV * g1[...]) + bb1[...]
    t = jnp.maximum(t, 0.0)
    t = jnp.dot(t, w2[...], preferred_element_type=jnp.float32) + b2[...]
    t = t * (_BN_INV * g2[...]) + bb2[...]
    if not last:
        t = jnp.maximum(t, 0.0)
        out[0] = t[:, :128]
        out[1] = t[:, 128:]
    else:
        out[...] = t


def _tc_post(hin2, agg2, eps_l, w1, b1, g1, bb1, w2, b2, g2, bb2, last):
    if last:
        out_spec = pl.BlockSpec((_BN_ROWS, D), lambda i: (i, 0))
        out_shape = jax.ShapeDtypeStruct((N, D), jnp.float32)
    else:
        out_spec = pl.BlockSpec((NC, _BN_ROWS, 128), lambda i: (0, i, 0))
        out_shape = jax.ShapeDtypeStruct((NC, N, 128), jnp.float32)
    return pl.pallas_call(
        functools.partial(_post_body, last),
        grid=(N // _BN_ROWS,),
        in_specs=[
            pl.BlockSpec((NC, _BN_ROWS, 128), lambda i: (0, i, 0)),
            pl.BlockSpec((NC, _BN_ROWS, 128), lambda i: (0, i, 0)),
            pl.BlockSpec((1, 1), lambda i: (0, 0)),
            pl.BlockSpec((D, 2 * D), lambda i: (0, 0)),
            pl.BlockSpec((1, 2 * D), lambda i: (0, 0)),
            pl.BlockSpec((1, 2 * D), lambda i: (0, 0)),
            pl.BlockSpec((1, 2 * D), lambda i: (0, 0)),
            pl.BlockSpec((2 * D, D), lambda i: (0, 0)),
            pl.BlockSpec((1, D), lambda i: (0, 0)),
            pl.BlockSpec((1, D), lambda i: (0, 0)),
            pl.BlockSpec((1, D), lambda i: (0, 0)),
        ],
        out_specs=out_spec,
        out_shape=out_shape,
    )(hin2, agg2, eps_l, w1, b1, g1, bb1, w2, b2, g2, bb2)


def _vn_body(pooled, vn, w1, b1, g1, bb1, w2, b2, g2, bb2, out):
    v = pooled[...] + vn[...]
    v = jnp.dot(v, w1[...], preferred_element_type=jnp.float32) + b1[...]
    v = v * (_BN_INV * g1[...]) + bb1[...]
    v = jnp.maximum(v, 0.0)
    v = jnp.dot(v, w2[...], preferred_element_type=jnp.float32) + b2[...]
    v = v * (_BN_INV * g2[...]) + bb2[...]
    out[...] = jnp.maximum(v, 0.0)


def _tc_vn(pooled, vn, w1, b1, g1, bb1, w2, b2, g2, bb2):
    return pl.pallas_call(
        _vn_body,
        out_shape=jax.ShapeDtypeStruct((G, D), jnp.float32),
    )(pooled, vn, w1, b1, g1, bb1, w2, b2, g2, bb2)


def kernel(x, edge_attr, edge_W, edge_b, eps, mlp_W1, mlp_b1, mlp_bn_g,
           mlp_bn_b, mlp_W2, mlp_b2, bn_g, bn_b, vn_emb, vW1, vb1, vbn1_g,
           vbn1_b, vW2, vb2, vbn2_g, vbn2_b, edge_index, batch):
    L = edge_W.shape[0]
    h2 = jnp.stack([x[:, :128], x[:, 128:]], axis=0)
    ea8 = jnp.pad(edge_attr, ((0, 0), (0, 1)))
    w8 = jnp.pad(edge_W, ((0, 0), (0, 1), (0, 0)))
    ridx = edge_index[0].reshape(NS, NCHUNK, CHUNK)
    col = edge_index[1]
    cidx2 = jnp.stack([jnp.where(col < NHALF, col, DUMP),
                       jnp.where(col >= NHALF, col - NHALF, DUMP)],
                      axis=0).reshape(2, NS, NCHUNK, CHUNK)
    batch2 = batch.reshape(N, 1)
    vn = jnp.tile(vn_emb[None, :], (G, 1))

    def r1(a):
        return a.reshape(1, -1)

    out = None
    for l in range(L):
        hin2, pooled = _tc_pre(h2, vn, batch2)
        ee2 = _tc_ee(ea8, w8[l], r1(edge_b[l]))
        agg2 = _sc_edge(hin2, ee2, ridx, cidx2)
        last = l == L - 1
        res = _tc_post(hin2, agg2, eps[l].reshape(1, 1),
                       mlp_W1[l], r1(mlp_b1[l]), r1(mlp_bn_g[l]),
                       r1(mlp_bn_b[l]), mlp_W2[l], r1(mlp_b2[l]),
                       r1(bn_g[l]), r1(bn_b[l]), last)
        if last:
            out = res
        else:
            h2 = res
            vn = _tc_vn(pooled, vn, vW1[l], r1(vb1[l]), r1(vbn1_g[l]),
                        r1(vbn1_b[l]), vW2[l], r1(vb2[l]), r1(vbn2_g[l]),
                        r1(vbn2_b[l]))
    return out


# speedup vs baseline: 2.7788x; 1.1111x over previous
import functools
import math

import jax
import jax.numpy as jnp
from jax import lax
from jax.experimental import pallas as pl
from jax.experimental.pallas import tpu as pltpu
from jax.experimental.pallas import tpu_sc as plsc

N = 10000
E = 160000
D = 256
G = 64
NC = 2
NS = 16
CHUNK = 40
EPW = E // NS
NCHUNK = EPW // CHUNK
NPAD = 10240
NHALF = 5120
ACC_ROWS = 5248
DUMP = 5120
ZSTRIPE = ACC_ROWS // NS
OSTRIPE = NHALF // NS
ZROWS = 8

_BN_INV = float(1.0 / math.sqrt(1.0 + 1e-5))


QBLKS = ((0, 64), (64, 64), (128, 64), (192, 58))


def _sc_edge_body(h2, ee2, ridx, cidx2, agg2, idxr_v, idxc_v,
                  hbuf0, hbuf1, hbuf2, eebuf0, eebuf1, eebuf2,
                  zbuf, acc, gsem0, gsem1, gsem2,
                  esem0, esem1, esem2, ssem0, ssem1, ssem2):
    c = lax.axis_index("c")
    s = lax.axis_index("s")
    hbufs = (hbuf0, hbuf1, hbuf2)
    eebufs = (eebuf0, eebuf1, eebuf2)
    gsems = (gsem0, gsem1, gsem2)
    esems = (esem0, esem1, esem2)
    ssems = (ssem0, ssem1, ssem2)
    NSLOT = 3

    z16 = jnp.zeros((16,), jnp.float32)
    for i in range(ZROWS):
        for d in range(8):
            zbuf[i, pl.ds(16 * d, 16)] = z16

    def issue(q0, j, k):
        pltpu.async_copy(h2.at[c].at[idxr_v.at[j]], hbufs[k], gsems[k])
        e0 = pl.multiple_of(s * EPW + (q0 + j) * CHUNK, 8)
        pltpu.async_copy(ee2.at[c, pl.ds(e0, CHUNK)], eebufs[k], esems[k])

    def process(q0, j, k):
        pltpu.make_async_copy(h2.at[c].at[idxr_v.at[j]], hbufs[k],
                              gsems[k]).wait()
        e0 = pl.multiple_of(s * EPW + (q0 + j) * CHUNK, 8)
        pltpu.make_async_copy(ee2.at[c, pl.ds(e0, CHUNK)], eebufs[k],
                              esems[k]).wait()
        hb, eb = hbufs[k], eebufs[k]

        def row_body(r4, carry):
            for u in range(4):
                r = r4 * 4 + u
                for d in range(8):
                    sl = pl.ds(16 * d, 16)
                    hb[r, sl] = jnp.maximum(hb[r, sl] + eb[r, sl], 0.0)
            return carry

        lax.fori_loop(0, CHUNK // 4, row_body, 0)
        pltpu.async_copy(hb, acc.at[idxc_v.at[j]], ssems[k], add=True)

    def drain_scatter(j, k):
        pltpu.make_async_copy(hbufs[k], acc.at[idxc_v.at[j]],
                              ssems[k]).wait()

    for p in range(2):
        for j in range(ZSTRIPE // ZROWS):
            z0 = pl.multiple_of(s * ZSTRIPE + j * ZROWS, ZROWS)
            pltpu.sync_copy(zbuf, acc.at[pl.ds(z0, ZROWS)])
        plsc.subcore_barrier()

        for q0, qn in QBLKS:
            pltpu.sync_copy(ridx.at[s, pl.ds(q0, qn)],
                            idxr_v.at[pl.ds(0, qn)])
            pltpu.sync_copy(cidx2.at[p, s, pl.ds(q0, qn)],
                            idxc_v.at[pl.ds(0, qn)])
            for k in range(NSLOT):
                issue(q0, k, k)
            base = (qn // NSLOT) * NSLOT

            def tri_body(i, carry, q0=q0, base=base):
                j = i * NSLOT
                for u in range(NSLOT):
                    process(q0, j + u, u)
                    drain_scatter(j + u, u)

                    @pl.when(j + u + NSLOT < base)
                    def _():
                        issue(q0, j + u + NSLOT, u)

                return carry

            lax.fori_loop(0, qn // NSLOT, tri_body, 0)
            for u in range(qn - base):
                issue(q0, base + u, u)
            for u in range(qn - base):
                process(q0, base + u, u)
                drain_scatter(base + u, u)

        plsc.subcore_barrier()
        r0 = pl.multiple_of(s * OSTRIPE, 8)
        pltpu.sync_copy(acc.at[pl.ds(r0, OSTRIPE)],
                        agg2.at[c, pl.ds(p * NHALF + r0, OSTRIPE)])
        plsc.subcore_barrier()


def _sc_edge(hin2, ee2, ridx, cidx2):
    mesh = plsc.VectorSubcoreMesh(core_axis_name="c", subcore_axis_name="s",
                                  num_cores=NC, num_subcores=NS)
    f = pl.kernel(
        _sc_edge_body,
        out_type=jax.ShapeDtypeStruct((NC, NPAD, 128), jnp.float32),
        mesh=mesh,
        scratch_types=[
            pltpu.VMEM((64, CHUNK), jnp.int32),
            pltpu.VMEM((64, CHUNK), jnp.int32),
            pltpu.VMEM((CHUNK, 128), jnp.float32),
            pltpu.VMEM((CHUNK, 128), jnp.float32),
            pltpu.VMEM((CHUNK, 128), jnp.float32),
            pltpu.VMEM((CHUNK, 128), jnp.float32),
            pltpu.VMEM((CHUNK, 128), jnp.float32),
            pltpu.VMEM((CHUNK, 128), jnp.float32),
            pltpu.VMEM((ZROWS, 128), jnp.float32),
            pltpu.VMEM_SHARED((ACC_ROWS, 128), jnp.float32),
        ] + [pltpu.SemaphoreType.DMA] * 9,
    )
    return f(hin2, ee2, ridx, cidx2)


_BN_ROWS = 2000


def _pre_body(h2, vn, batch, hin2, pooled):
    i = pl.program_id(0)
    h = jnp.concatenate([h2[0], h2[1]], axis=1)
    oh = (batch[...] == lax.broadcasted_iota(jnp.int32, (1, G), 1)
          ).astype(jnp.float32)
    hin = h + jnp.dot(oh, vn[...], preferred_element_type=jnp.float32)
    hin2[0] = hin[:, :128]
    hin2[1] = hin[:, 128:]
    p = lax.dot_general(oh, hin, (((0,), (0,)), ((), ())),
                        preferred_element_type=jnp.float32)

    @pl.when(i == 0)
    def _():
        pooled[...] = p

    @pl.when(i > 0)
    def _():
        pooled[...] += p


def _tc_pre(h2, vn, batch2):
    return pl.pallas_call(
        _pre_body,
        grid=(N // _BN_ROWS,),
        in_specs=[
            pl.BlockSpec((NC, _BN_ROWS, 128), lambda i: (0, i, 0)),
            pl.BlockSpec((G, D), lambda i: (0, 0)),
            pl.BlockSpec((_BN_ROWS, 1), lambda i: (i, 0)),
        ],
        out_specs=[
            pl.BlockSpec((NC, _BN_ROWS, 128), lambda i: (0, i, 0)),
            pl.BlockSpec((G, D), lambda i: (0, 0)),
        ],
        out_shape=[
            jax.ShapeDtypeStruct((NC, N, 128), jnp.float32),
            jax.ShapeDtypeStruct((G, D), jnp.float32),
        ],
    )(h2, vn, batch2)


_EE_ROWS = 4000


def _ee_body(ea, w, b, out):
    t = jnp.dot(ea[...], w[...], preferred_element_type=jnp.float32) + b[...]
    out[0] = t[:, :128]
    out[1] = t[:, 128:]


def _tc_ee(ea8, w8, b):
    return pl.pallas_call(
        _ee_body,
        grid=(E // _EE_ROWS,),
        in_specs=[
            pl.BlockSpec((_EE_ROWS, 8), lambda i: (i, 0)),
            pl.BlockSpec((8, D), lambda i: (0, 0)),
            pl.BlockSpec((1, D), lambda i: (0, 0)),
        ],
        out_specs=pl.BlockSpec((NC, _EE_ROWS, 128), lambda i: (0, i, 0)),
        out_shape=jax.ShapeDtypeStruct((NC, E, 128), jnp.float32),
    )(ea8, w8, b)


def _post_body(last, hin2, agg2, eps, w1, b1, g1, bb1, w2, b2, g2, bb2, out):
    hin = jnp.concatenate([hin2[0], hin2[1]], axis=1)
    agg = jnp.concatenate([agg2[0], agg2[1]], axis=1)
    t = (1.0 + eps[0, 0]) * hin + agg
    t = jnp.dot(t, w1[...], preferred_element_type=jnp.float32) + b1[...]
    t = t * (_BN_INV * g1[...]) + bb1[...]
    t = jnp.maximum(t, 0.0)
    t = jnp.dot(t, w2[...], preferred_element_type=jnp.float32) + b2[...]
    t = t * (_BN_INV * g2[...]) + bb2[...]
    if not last:
        t = jnp.maximum(t, 0.0)
        out[0] = t[:, :128]
        out[1] = t[:, 128:]
    else:
        out[...] = t


def _tc_post(hin2, agg2, eps_l, w1, b1, g1, bb1, w2, b2, g2, bb2, last):
    if last:
        out_spec = pl.BlockSpec((_BN_ROWS, D), lambda i: (i, 0))
        out_shape = jax.ShapeDtypeStruct((N, D), jnp.float32)
    else:
        out_spec = pl.BlockSpec((NC, _BN_ROWS, 128), lambda i: (0, i, 0))
        out_shape = jax.ShapeDtypeStruct((NC, N, 128), jnp.float32)
    return pl.pallas_call(
        functools.partial(_post_body, last),
        grid=(N // _BN_ROWS,),
        in_specs=[
            pl.BlockSpec((NC, _BN_ROWS, 128), lambda i: (0, i, 0)),
            pl.BlockSpec((NC, _BN_ROWS, 128), lambda i: (0, i, 0)),
            pl.BlockSpec((1, 1), lambda i: (0, 0)),
            pl.BlockSpec((D, 2 * D), lambda i: (0, 0)),
            pl.BlockSpec((1, 2 * D), lambda i: (0, 0)),
            pl.BlockSpec((1, 2 * D), lambda i: (0, 0)),
            pl.BlockSpec((1, 2 * D), lambda i: (0, 0)),
            pl.BlockSpec((2 * D, D), lambda i: (0, 0)),
            pl.BlockSpec((1, D), lambda i: (0, 0)),
            pl.BlockSpec((1, D), lambda i: (0, 0)),
            pl.BlockSpec((1, D), lambda i: (0, 0)),
        ],
        out_specs=out_spec,
        out_shape=out_shape,
    )(hin2, agg2, eps_l, w1, b1, g1, bb1, w2, b2, g2, bb2)


def _vn_body(pooled, vn, w1, b1, g1, bb1, w2, b2, g2, bb2, out):
    v = pooled[...] + vn[...]
    v = jnp.dot(v, w1[...], preferred_element_type=jnp.float32) + b1[...]
    v = v * (_BN_INV * g1[...]) + bb1[...]
    v = jnp.maximum(v, 0.0)
    v = jnp.dot(v, w2[...], preferred_element_type=jnp.float32) + b2[...]
    v = v * (_BN_INV * g2[...]) + bb2[...]
    out[...] = jnp.maximum(v, 0.0)


def _tc_vn(pooled, vn, w1, b1, g1, bb1, w2, b2, g2, bb2):
    return pl.pallas_call(
        _vn_body,
        out_shape=jax.ShapeDtypeStruct((G, D), jnp.float32),
    )(pooled, vn, w1, b1, g1, bb1, w2, b2, g2, bb2)


def kernel(x, edge_attr, edge_W, edge_b, eps, mlp_W1, mlp_b1, mlp_bn_g,
           mlp_bn_b, mlp_W2, mlp_b2, bn_g, bn_b, vn_emb, vW1, vb1, vbn1_g,
           vbn1_b, vW2, vb2, vbn2_g, vbn2_b, edge_index, batch):
    L = edge_W.shape[0]
    h2 = jnp.stack([x[:, :128], x[:, 128:]], axis=0)
    ea8 = jnp.pad(edge_attr, ((0, 0), (0, 1)))
    w8 = jnp.pad(edge_W, ((0, 0), (0, 1), (0, 0)))
    ridx = edge_index[0].reshape(NS, NCHUNK, CHUNK)
    col = edge_index[1]
    cidx2 = jnp.stack([jnp.where(col < NHALF, col, DUMP),
                       jnp.where(col >= NHALF, col - NHALF, DUMP)],
                      axis=0).reshape(2, NS, NCHUNK, CHUNK)
    batch2 = batch.reshape(N, 1)
    vn = jnp.tile(vn_emb[None, :], (G, 1))

    def r1(a):
        return a.reshape(1, -1)

    out = None
    for l in range(L):
        hin2, pooled = _tc_pre(h2, vn, batch2)
        ee2 = _tc_ee(ea8, w8[l], r1(edge_b[l]))
        agg2 = _sc_edge(hin2, ee2, ridx, cidx2)
        last = l == L - 1
        res = _tc_post(hin2, agg2, eps[l].reshape(1, 1),
                       mlp_W1[l], r1(mlp_b1[l]), r1(mlp_bn_g[l]),
                       r1(mlp_bn_b[l]), mlp_W2[l], r1(mlp_b2[l]),
                       r1(bn_g[l]), r1(bn_b[l]), last)
        if last:
            out = res
        else:
            h2 = res
            vn = _tc_vn(pooled, vn, vW1[l], r1(vb1[l]), r1(vbn1_g[l]),
                        r1(vbn1_b[l]), vW2[l], r1(vb2[l]), r1(vbn2_g[l]),
                        r1(vbn2_b[l]))
    return out


# speedup vs baseline: 2.7825x; 1.0013x over previous
import functools
import math

import jax
import jax.numpy as jnp
from jax import lax
from jax.experimental import pallas as pl
from jax.experimental.pallas import tpu as pltpu
from jax.experimental.pallas import tpu_sc as plsc

N = 10000
E = 160000
D = 256
G = 64
NC = 2
NS = 16
CHUNK = 40
EPW = E // NS
NCHUNK = EPW // CHUNK
NPAD = 10240
NHALF = 5120
ACC_ROWS = 5248
DUMP = 5120
ZSTRIPE = ACC_ROWS // NS
OSTRIPE = NHALF // NS
ZROWS = 8

_BN_INV = float(1.0 / math.sqrt(1.0 + 1e-5))


QBLKS = ((0, 64), (64, 64), (128, 64), (192, 58))


def _sc_edge_body(h2, ee2, ridx, cidx2, agg2, idxr_v, idxc_v,
                  hbuf0, hbuf1, hbuf2, hbuf3, eebuf0, eebuf1, eebuf2, eebuf3,
                  zbuf, acc, gsem0, gsem1, gsem2, gsem3,
                  esem0, esem1, esem2, esem3, ssem0, ssem1, ssem2, ssem3):
    c = lax.axis_index("c")
    s = lax.axis_index("s")
    hbufs = (hbuf0, hbuf1, hbuf2, hbuf3)
    eebufs = (eebuf0, eebuf1, eebuf2, eebuf3)
    gsems = (gsem0, gsem1, gsem2, gsem3)
    esems = (esem0, esem1, esem2, esem3)
    ssems = (ssem0, ssem1, ssem2, ssem3)
    NSLOT = 4

    z16 = jnp.zeros((16,), jnp.float32)
    for i in range(ZROWS):
        for d in range(8):
            zbuf[i, pl.ds(16 * d, 16)] = z16

    def issue(q0, j, k):
        pltpu.async_copy(h2.at[c].at[idxr_v.at[j]], hbufs[k], gsems[k])
        e0 = pl.multiple_of(s * EPW + (q0 + j) * CHUNK, 8)
        pltpu.async_copy(ee2.at[c, pl.ds(e0, CHUNK)], eebufs[k], esems[k])

    def process(q0, j, k):
        pltpu.make_async_copy(h2.at[c].at[idxr_v.at[j]], hbufs[k],
                              gsems[k]).wait()
        e0 = pl.multiple_of(s * EPW + (q0 + j) * CHUNK, 8)
        pltpu.make_async_copy(ee2.at[c, pl.ds(e0, CHUNK)], eebufs[k],
                              esems[k]).wait()
        hb, eb = hbufs[k], eebufs[k]

        def row_body(r4, carry):
            for u in range(4):
                r = r4 * 4 + u
                for d in range(8):
                    sl = pl.ds(16 * d, 16)
                    hb[r, sl] = jnp.maximum(hb[r, sl] + eb[r, sl], 0.0)
            return carry

        lax.fori_loop(0, CHUNK // 4, row_body, 0)
        pltpu.async_copy(hb, acc.at[idxc_v.at[j]], ssems[k], add=True)

    def drain_scatter(j, k):
        pltpu.make_async_copy(hbufs[k], acc.at[idxc_v.at[j]],
                              ssems[k]).wait()

    for p in range(2):
        for j in range(ZSTRIPE // ZROWS):
            z0 = pl.multiple_of(s * ZSTRIPE + j * ZROWS, ZROWS)
            pltpu.sync_copy(zbuf, acc.at[pl.ds(z0, ZROWS)])
        plsc.subcore_barrier()

        for q0, qn in QBLKS:
            pltpu.sync_copy(ridx.at[s, pl.ds(q0, qn)],
                            idxr_v.at[pl.ds(0, qn)])
            pltpu.sync_copy(cidx2.at[p, s, pl.ds(q0, qn)],
                            idxc_v.at[pl.ds(0, qn)])
            for k in range(NSLOT):
                issue(q0, k, k)
            base = (qn // NSLOT) * NSLOT

            def tri_body(i, carry, q0=q0, base=base):
                j = i * NSLOT
                for u in range(NSLOT):
                    process(q0, j + u, u)
                    drain_scatter(j + u, u)

                    @pl.when(j + u + NSLOT < base)
                    def _():
                        issue(q0, j + u + NSLOT, u)

                return carry

            lax.fori_loop(0, qn // NSLOT, tri_body, 0)
            for u in range(qn - base):
                issue(q0, base + u, u)
            for u in range(qn - base):
                process(q0, base + u, u)
                drain_scatter(base + u, u)

        plsc.subcore_barrier()
        r0 = pl.multiple_of(s * OSTRIPE, 8)
        pltpu.sync_copy(acc.at[pl.ds(r0, OSTRIPE)],
                        agg2.at[c, pl.ds(p * NHALF + r0, OSTRIPE)])
        plsc.subcore_barrier()


def _sc_edge(hin2, ee2, ridx, cidx2):
    mesh = plsc.VectorSubcoreMesh(core_axis_name="c", subcore_axis_name="s",
                                  num_cores=NC, num_subcores=NS)
    f = pl.kernel(
        _sc_edge_body,
        out_type=jax.ShapeDtypeStruct((NC, NPAD, 128), jnp.float32),
        mesh=mesh,
        scratch_types=[
            pltpu.VMEM((64, CHUNK), jnp.int32),
            pltpu.VMEM((64, CHUNK), jnp.int32),
            pltpu.VMEM((CHUNK, 128), jnp.float32),
            pltpu.VMEM((CHUNK, 128), jnp.float32),
            pltpu.VMEM((CHUNK, 128), jnp.float32),
            pltpu.VMEM((CHUNK, 128), jnp.float32),
            pltpu.VMEM((CHUNK, 128), jnp.float32),
            pltpu.VMEM((CHUNK, 128), jnp.float32),
            pltpu.VMEM((CHUNK, 128), jnp.float32),
            pltpu.VMEM((CHUNK, 128), jnp.float32),
            pltpu.VMEM((ZROWS, 128), jnp.float32),
            pltpu.VMEM_SHARED((ACC_ROWS, 128), jnp.float32),
        ] + [pltpu.SemaphoreType.DMA] * 12,
    )
    return f(hin2, ee2, ridx, cidx2)


_BN_ROWS = 2000


def _pre_body(h2, vn, batch, hin2, pooled):
    i = pl.program_id(0)
    h = jnp.concatenate([h2[0], h2[1]], axis=1)
    oh = (batch[...] == lax.broadcasted_iota(jnp.int32, (1, G), 1)
          ).astype(jnp.float32)
    hin = h + jnp.dot(oh, vn[...], preferred_element_type=jnp.float32)
    hin2[0] = hin[:, :128]
    hin2[1] = hin[:, 128:]
    p = lax.dot_general(oh, hin, (((0,), (0,)), ((), ())),
                        preferred_element_type=jnp.float32)

    @pl.when(i == 0)
    def _():
        pooled[...] = p

    @pl.when(i > 0)
    def _():
        pooled[...] += p


def _tc_pre(h2, vn, batch2):
    return pl.pallas_call(
        _pre_body,
        grid=(N // _BN_ROWS,),
        in_specs=[
            pl.BlockSpec((NC, _BN_ROWS, 128), lambda i: (0, i, 0)),
            pl.BlockSpec((G, D), lambda i: (0, 0)),
            pl.BlockSpec((_BN_ROWS, 1), lambda i: (i, 0)),
        ],
        out_specs=[
            pl.BlockSpec((NC, _BN_ROWS, 128), lambda i: (0, i, 0)),
            pl.BlockSpec((G, D), lambda i: (0, 0)),
        ],
        out_shape=[
            jax.ShapeDtypeStruct((NC, N, 128), jnp.float32),
            jax.ShapeDtypeStruct((G, D), jnp.float32),
        ],
    )(h2, vn, batch2)


_EE_ROWS = 4000


def _ee_body(ea, w, b, out):
    t = jnp.dot(ea[...], w[...], preferred_element_type=jnp.float32) + b[...]
    out[0] = t[:, :128]
    out[1] = t[:, 128:]


def _tc_ee(ea8, w8, b):
    return pl.pallas_call(
        _ee_body,
        grid=(E // _EE_ROWS,),
        in_specs=[
            pl.BlockSpec((_EE_ROWS, 8), lambda i: (i, 0)),
            pl.BlockSpec((8, D), lambda i: (0, 0)),
            pl.BlockSpec((1, D), lambda i: (0, 0)),
        ],
        out_specs=pl.BlockSpec((NC, _EE_ROWS, 128), lambda i: (0, i, 0)),
        out_shape=jax.ShapeDtypeStruct((NC, E, 128), jnp.float32),
    )(ea8, w8, b)


def _post_body(last, hin2, agg2, eps, w1, b1, g1, bb1, w2, b2, g2, bb2, out):
    hin = jnp.concatenate([hin2[0], hin2[1]], axis=1)
    agg = jnp.concatenate([agg2[0], agg2[1]], axis=1)
    t = (1.0 + eps[0, 0]) * hin + agg
    t = jnp.dot(t, w1[...], preferred_element_type=jnp.float32) + b1[...]
    t = t * (_BN_INV * g1[...]) + bb1[...]
    t = jnp.maximum(t, 0.0)
    t = jnp.dot(t, w2[...], preferred_element_type=jnp.float32) + b2[...]
    t = t * (_BN_INV * g2[...]) + bb2[...]
    if not last:
        t = jnp.maximum(t, 0.0)
        out[0] = t[:, :128]
        out[1] = t[:, 128:]
    else:
        out[...] = t


def _tc_post(hin2, agg2, eps_l, w1, b1, g1, bb1, w2, b2, g2, bb2, last):
    if last:
        out_spec = pl.BlockSpec((_BN_ROWS, D), lambda i: (i, 0))
        out_shape = jax.ShapeDtypeStruct((N, D), jnp.float32)
    else:
        out_spec = pl.BlockSpec((NC, _BN_ROWS, 128), lambda i: (0, i, 0))
        out_shape = jax.ShapeDtypeStruct((NC, N, 128), jnp.float32)
    return pl.pallas_call(
        functools.partial(_post_body, last),
        grid=(N // _BN_ROWS,),
        in_specs=[
            pl.BlockSpec((NC, _BN_ROWS, 128), lambda i: (0, i, 0)),
            pl.BlockSpec((NC, _BN_ROWS, 128), lambda i: (0, i, 0)),
            pl.BlockSpec((1, 1), lambda i: (0, 0)),
            pl.BlockSpec((D, 2 * D), lambda i: (0, 0)),
            pl.BlockSpec((1, 2 * D), lambda i: (0, 0)),
            pl.BlockSpec((1, 2 * D), lambda i: (0, 0)),
            pl.BlockSpec((1, 2 * D), lambda i: (0, 0)),
            pl.BlockSpec((2 * D, D), lambda i: (0, 0)),
            pl.BlockSpec((1, D), lambda i: (0, 0)),
            pl.BlockSpec((1, D), lambda i: (0, 0)),
            pl.BlockSpec((1, D), lambda i: (0, 0)),
        ],
        out_specs=out_spec,
        out_shape=out_shape,
    )(hin2, agg2, eps_l, w1, b1, g1, bb1, w2, b2, g2, bb2)


def _vn_body(pooled, vn, w1, b1, g1, bb1, w2, b2, g2, bb2, out):
    v = pooled[...] + vn[...]
    v = jnp.dot(v, w1[...], preferred_element_type=jnp.float32) + b1[...]
    v = v * (_BN_INV * g1[...]) + bb1[...]
    v = jnp.maximum(v, 0.0)
    v = jnp.dot(v, w2[...], preferred_element_type=jnp.float32) + b2[...]
    v = v * (_BN_INV * g2[...]) + bb2[...]
    out[...] = jnp.maximum(v, 0.0)


def _tc_vn(pooled, vn, w1, b1, g1, bb1, w2, b2, g2, bb2):
    return pl.pallas_call(
        _vn_body,
        out_shape=jax.ShapeDtypeStruct((G, D), jnp.float32),
    )(pooled, vn, w1, b1, g1, bb1, w2, b2, g2, bb2)


def kernel(x, edge_attr, edge_W, edge_b, eps, mlp_W1, mlp_b1, mlp_bn_g,
           mlp_bn_b, mlp_W2, mlp_b2, bn_g, bn_b, vn_emb, vW1, vb1, vbn1_g,
           vbn1_b, vW2, vb2, vbn2_g, vbn2_b, edge_index, batch):
    L = edge_W.shape[0]
    h2 = jnp.stack([x[:, :128], x[:, 128:]], axis=0)
    ea8 = jnp.pad(edge_attr, ((0, 0), (0, 1)))
    w8 = jnp.pad(edge_W, ((0, 0), (0, 1), (0, 0)))
    ridx = edge_index[0].reshape(NS, NCHUNK, CHUNK)
    col = edge_index[1]
    cidx2 = jnp.stack([jnp.where(col < NHALF, col, DUMP),
                       jnp.where(col >= NHALF, col - NHALF, DUMP)],
                      axis=0).reshape(2, NS, NCHUNK, CHUNK)
    batch2 = batch.reshape(N, 1)
    vn = jnp.tile(vn_emb[None, :], (G, 1))

    def r1(a):
        return a.reshape(1, -1)

    out = None
    for l in range(L):
        hin2, pooled = _tc_pre(h2, vn, batch2)
        ee2 = _tc_ee(ea8, w8[l], r1(edge_b[l]))
        agg2 = _sc_edge(hin2, ee2, ridx, cidx2)
        last = l == L - 1
        res = _tc_post(hin2, agg2, eps[l].reshape(1, 1),
                       mlp_W1[l], r1(mlp_b1[l]), r1(mlp_bn_g[l]),
                       r1(mlp_bn_b[l]), mlp_W2[l], r1(mlp_b2[l]),
                       r1(bn_g[l]), r1(bn_b[l]), last)
        if last:
            out = res
        else:
            h2 = res
            vn = _tc_vn(pooled, vn, vW1[l], r1(vb1[l]), r1(vbn1_g[l]),
                        r1(vbn1_b[l]), vW2[l], r1(vb2[l]), r1(vbn2_g[l]),
                        r1(vbn2_b[l]))
    return out
